# baseline (speedup 1.0000x reference)
import functools

import jax
import jax.numpy as jnp
from jax import lax
from jax.experimental import pallas as pl
from jax.experimental.pallas import tpu as pltpu
from jax.experimental.pallas import tpu_sc as plsc

_NUM_USERS = 30000
_NUM_ITEMS = 70000
_N = _NUM_USERS + _NUM_ITEMS
_H = 16
_E = 1600000
_NS = 16
_SUB = 128
_CHUNK = 384
_K = _CHUNK // _SUB
_GROUP = 3
_GROWS = _GROUP * _K
_NMACRO = 30
_NCHUNK = 9 * _NMACRO
_NGROUP = _NCHUNK // _GROUP
_EPT = _NCHUNK * _CHUNK
_EPAD = _EPT * _NS
_IROWS_PT = _EPT // _SUB
_NP = 100352
_RPT = _NP // _NS
_QROWS = 224
_NQ = _RPT // _QROWS


def _spmm3_body(x0, edata, sum_o, xbuf,
                g0, g1, g2, e0, e1, e2, acc,
                sg0, sg1, sg2, ss0, ss1, ss2, se0, se1, se2):
    c = lax.axis_index("c")
    s = lax.axis_index("s")
    r0 = s * _RPT
    gs = (g0, g1, g2)
    es = (e0, e1, e2)
    sgs = (sg0, sg1, sg2)
    sss = (ss0, ss1, ss2)
    ses = (se0, se1, se2)
    sa_v = g0.at[pl.ds(0, _QROWS)]
    sb_v = g1.at[pl.ds(0, _QROWS)]

    def fire_load_e(gt, eb):
        pltpu.async_copy(
            edata.at[pl.ds(s * _IROWS_PT + gt * _GROWS, _GROWS)], es[eb],
            ses[eb])

    def wait_e(eb):
        pltpu.make_async_copy(edata.at[pl.ds(0, _GROWS)], es[eb],
                              ses[eb]).wait()

    def fire_gather(cq, eb, p):
        for j in range(_K):
            pltpu.async_copy(xbuf.at[c].at[es[eb].at[_K * cq + j, 0]],
                             gs[p].at[pl.ds(j * _SUB, _SUB)], sgs[p])

    def drain(sem):
        for j in range(_K):
            pltpu.make_async_copy(xbuf.at[c].at[pl.ds(0, _SUB)],
                                  gs[0].at[pl.ds(0, _SUB)], sem).wait()

    def scale(cq, eb, p):
        @plsc.parallel_loop(0, _CHUNK // 16, unroll=2)
        def _(mg):
            m0 = mg * 16
            j = mg // (_SUB // 16)
            mm = (mg % (_SUB // 16)) * 16
            vv = plsc.bitcast(es[eb][_K * cq + j, 2, pl.ds(mm, 16)],
                              jnp.float32)
            for t in range(16):
                gs[p][m0 + t, :] = gs[p][m0 + t, :] * vv[t]

    def fire_scatter(cq, eb, p):
        for j in range(_K):
            pltpu.async_copy(gs[p].at[pl.ds(j * _SUB, _SUB)],
                             acc.at[es[eb].at[_K * cq + j, 1]], sss[p],
                             add=True)

    def pre(q, _):
        off = r0 + q * _QROWS
        pltpu.sync_copy(x0.at[c].at[pl.ds(off, _QROWS)], sa_v)
        pltpu.sync_copy(sa_v, xbuf.at[c].at[pl.ds(off, _QROWS)])
        pltpu.sync_copy(sa_v, sum_o.at[c].at[pl.ds(off, _QROWS)])
        return 0
    lax.fori_loop(0, _NQ, pre, 0)

    def layer(l, _):
        mult = jnp.where(l == 2, jnp.float32(0.25), jnp.float32(1.0))

        def zrow(i, _):
            sa_v[i, :] = jnp.zeros((_H,), jnp.float32)
            return 0
        lax.fori_loop(0, _QROWS, zrow, 0)

        def zq(q, _):
            pltpu.sync_copy(sa_v, acc.at[pl.ds(r0 + q * _QROWS, _QROWS)])
            return 0
        lax.fori_loop(0, _NQ, zq, 0)
        plsc.subcore_barrier()

        fire_load_e(0, 0)
        wait_e(0)
        fire_gather(0, 0, 0)

        def macro(t, _):
            for u in range(9):
                i = 9 * t + u
                p = u % 3
                pn = (u + 1) % 3
                eb = u // 3
                cq = u % 3

                @pl.when(i >= 2)
                def _():
                    drain(sss[pn])
                if cq == 0:
                    gnext = 3 * t + eb + 1

                    @pl.when(gnext < _NGROUP)
                    def _():
                        fire_load_e(gnext, (eb + 1) % 3)
                if cq == 2:
                    @pl.when(i + 1 < _NCHUNK)
                    def _():
                        wait_e((eb + 1) % 3)
                        fire_gather(0, (eb + 1) % 3, pn)
                else:
                    fire_gather(cq + 1, eb, pn)
                drain(sgs[p])
                scale(cq, eb, p)
                fire_scatter(cq, eb, p)
            return 0
        lax.fori_loop(0, _NMACRO, macro, 0)
        drain(sss[(_NCHUNK - 2) % 3])
        drain(sss[(_NCHUNK - 1) % 3])
        plsc.subcore_barrier()

        def cq_(q, _):
            off = r0 + q * _QROWS
            pltpu.sync_copy(acc.at[pl.ds(off, _QROWS)], sa_v)
            pltpu.sync_copy(sa_v, xbuf.at[c].at[pl.ds(off, _QROWS)])
            pltpu.sync_copy(sum_o.at[c].at[pl.ds(off, _QROWS)], sb_v)

            def addr(r, _):
                sb_v[r, :] = (sb_v[r, :] + sa_v[r, :]) * mult
                return 0
            lax.fori_loop(0, _QROWS, addr, 0)
            pltpu.sync_copy(sb_v, sum_o.at[c].at[pl.ds(off, _QROWS)])
            return 0
        lax.fori_loop(0, _NQ, cq_, 0)
        plsc.subcore_barrier()
        return 0
    lax.fori_loop(0, 3, layer, 0)


_spmm3 = functools.partial(
    pl.kernel,
    mesh=plsc.VectorSubcoreMesh(core_axis_name="c", subcore_axis_name="s"),
    compiler_params=pltpu.CompilerParams(use_tc_tiling_on_sc=False,
                                         needs_layout_passes=False),
    out_type=[
        jax.ShapeDtypeStruct((2, _NP, _H), jnp.float32),
        jax.ShapeDtypeStruct((2, _NP, _H), jnp.float32),
    ],
    scratch_types=[
        pltpu.VMEM((_CHUNK, _H), jnp.float32),
        pltpu.VMEM((_CHUNK, _H), jnp.float32),
        pltpu.VMEM((_CHUNK, _H), jnp.float32),
        pltpu.VMEM((_GROWS, 3, _SUB), jnp.int32),
        pltpu.VMEM((_GROWS, 3, _SUB), jnp.int32),
        pltpu.VMEM((_GROWS, 3, _SUB), jnp.int32),
        pltpu.VMEM_SHARED((_NP, _H), jnp.float32),
        pltpu.SemaphoreType.DMA,
        pltpu.SemaphoreType.DMA,
        pltpu.SemaphoreType.DMA,
        pltpu.SemaphoreType.DMA,
        pltpu.SemaphoreType.DMA,
        pltpu.SemaphoreType.DMA,
        pltpu.SemaphoreType.DMA,
        pltpu.SemaphoreType.DMA,
        pltpu.SemaphoreType.DMA,
    ],
)(_spmm3_body)


def kernel(user_emb, item_emb, edge_values, edge_index):
    all_emb = jnp.concatenate(
        [user_emb, item_emb, jnp.zeros((_NP - _N, 32), jnp.float32)], axis=0)
    x0 = jnp.stack([all_emb[:, :_H], all_emb[:, _H:]], axis=0)
    rows = edge_index[0].astype(jnp.int32)
    cols = edge_index[1].astype(jnp.int32)
    pad = _EPAD - _E
    cols_p = jnp.concatenate([cols, jnp.zeros((pad,), jnp.int32)]).reshape(-1, _SUB)
    rows_p = jnp.concatenate([rows, jnp.zeros((pad,), jnp.int32)]).reshape(-1, _SUB)
    vals_p = jnp.concatenate([edge_values, jnp.zeros((pad,), jnp.float32)]).reshape(-1, _SUB)
    vals_i = jax.lax.bitcast_convert_type(vals_p, jnp.int32)
    edata = jnp.stack([cols_p, rows_p, vals_i], axis=1)
    sum_o, _ = _spmm3(x0, edata)
    final = jnp.concatenate([sum_o[0, :_N], sum_o[1, :_N]], axis=1)
    return final[:_NUM_USERS], final[_NUM_USERS:]


# speedup vs baseline: 1.3648x; 1.3648x over previous
import functools

import jax
import jax.numpy as jnp
from jax import lax
from jax.experimental import pallas as pl
from jax.experimental.pallas import tpu as pltpu
from jax.experimental.pallas import tpu_sc as plsc

_NUM_USERS = 30000
_NUM_ITEMS = 70000
_N = _NUM_USERS + _NUM_ITEMS
_H = 16
_E = 1600000
_NS = 16
_SUB = 128
_CHUNK = 512
_K = _CHUNK // _SUB
_NMACRO = 66
_NCHUNK = 3 * _NMACRO
_EPT = _NCHUNK * _CHUNK
_EPAD = _EPT * _NS
_IROWS_PT = _EPT // _SUB
_NP = 100352
_RPT = _NP // _NS
_QROWS = 448
_NQ = _RPT // _QROWS


def _spmm3_body(x0, edata, sum_o, xbuf,
                g0, g1, g2, e0, e1, e2, acc,
                sg0, sg1, sg2, ss0, ss1, ss2, se0, se1, se2):
    c = lax.axis_index("c")
    s = lax.axis_index("s")
    r0 = s * _RPT
    gs = (g0, g1, g2)
    es = (e0, e1, e2)
    sgs = (sg0, sg1, sg2)
    sss = (ss0, ss1, ss2)
    ses = (se0, se1, se2)
    sa_v = g0.at[pl.ds(0, _QROWS)]
    sb_v = g1.at[pl.ds(0, _QROWS)]

    def load_e(i, p):
        pltpu.async_copy(edata.at[pl.ds(s * _IROWS_PT + i * _K, _K)], es[p],
                         ses[p])

    def wait_e(p):
        pltpu.make_async_copy(edata.at[pl.ds(0, _K)], es[p], ses[p]).wait()

    def fire_gather(src, p):
        for j in range(_K):
            pltpu.async_copy(src.at[c].at[es[p].at[j, 0]],
                             gs[p].at[pl.ds(j * _SUB, _SUB)], sgs[p])

    def drain(src, sem):
        for j in range(_K):
            pltpu.make_async_copy(src.at[c].at[pl.ds(0, _SUB)],
                                  gs[0].at[pl.ds(0, _SUB)], sem).wait()

    def scale(p):
        @plsc.parallel_loop(0, _CHUNK // 16, unroll=2)
        def _(mg):
            m0 = mg * 16
            j = mg // (_SUB // 16)
            mm = (mg % (_SUB // 16)) * 16
            vv = plsc.bitcast(es[p][j, 2, pl.ds(mm, 16)], jnp.float32)
            for t in range(16):
                gs[p][m0 + t, :] = gs[p][m0 + t, :] * vv[t]

    def fire_scatter(p):
        for j in range(_K):
            pltpu.async_copy(gs[p].at[pl.ds(j * _SUB, _SUB)],
                             acc.at[es[p].at[j, 1]], sss[p], add=True)

    for l in range(3):
        def zrow(i, _):
            sa_v[i, :] = jnp.zeros((_H,), jnp.float32)
            return 0
        lax.fori_loop(0, _QROWS, zrow, 0)

        def zq(q, _):
            pltpu.sync_copy(sa_v, acc.at[pl.ds(r0 + q * _QROWS, _QROWS)])
            return 0
        lax.fori_loop(0, _NQ, zq, 0)
        plsc.subcore_barrier()

        src = x0 if l == 0 else xbuf

        load_e(0, 0)
        wait_e(0)
        fire_gather(src, 0)

        def macro(t, _):
            for q in range(3):
                i = 3 * t + q
                pn = (q + 1) % 3
                @pl.when(i >= 2)
                def _():
                    drain(src, sss[pn])
                @pl.when(i + 1 < _NCHUNK)
                def _():
                    load_e(i + 1, pn)
                drain(src, sgs[q])
                @pl.when(i + 1 < _NCHUNK)
                def _():
                    wait_e(pn)
                    fire_gather(src, pn)
                scale(q)
                fire_scatter(q)
            return 0
        lax.fori_loop(0, _NMACRO, macro, 0)
        drain(src, sss[(_NCHUNK - 2) % 3])
        drain(src, sss[(_NCHUNK - 1) % 3])
        plsc.subcore_barrier()

        def cq(q, _):
            off = r0 + q * _QROWS
            pltpu.sync_copy(acc.at[pl.ds(off, _QROWS)], sa_v)
            if l < 2:
                pltpu.sync_copy(sa_v, xbuf.at[c].at[pl.ds(off, _QROWS)])
            prev = x0 if l == 0 else sum_o
            pltpu.sync_copy(prev.at[c].at[pl.ds(off, _QROWS)], sb_v)

            def addr(r, _):
                if l == 2:
                    sb_v[r, :] = (sb_v[r, :] + sa_v[r, :]) * 0.25
                else:
                    sb_v[r, :] = sb_v[r, :] + sa_v[r, :]
                return 0
            lax.fori_loop(0, _QROWS, addr, 0)
            pltpu.sync_copy(sb_v, sum_o.at[c].at[pl.ds(off, _QROWS)])
            return 0
        lax.fori_loop(0, _NQ, cq, 0)
        plsc.subcore_barrier()


_spmm3 = functools.partial(
    pl.kernel,
    mesh=plsc.VectorSubcoreMesh(core_axis_name="c", subcore_axis_name="s"),
    compiler_params=pltpu.CompilerParams(use_tc_tiling_on_sc=False,
                                         needs_layout_passes=False),
    out_type=[
        jax.ShapeDtypeStruct((2, _NP, _H), jnp.float32),
        jax.ShapeDtypeStruct((2, _NP, _H), jnp.float32),
    ],
    scratch_types=[
        pltpu.VMEM((_CHUNK, _H), jnp.float32),
        pltpu.VMEM((_CHUNK, _H), jnp.float32),
        pltpu.VMEM((_CHUNK, _H), jnp.float32),
        pltpu.VMEM((_K, 3, _SUB), jnp.int32),
        pltpu.VMEM((_K, 3, _SUB), jnp.int32),
        pltpu.VMEM((_K, 3, _SUB), jnp.int32),
        pltpu.VMEM_SHARED((_NP, _H), jnp.float32),
        pltpu.SemaphoreType.DMA,
        pltpu.SemaphoreType.DMA,
        pltpu.SemaphoreType.DMA,
        pltpu.SemaphoreType.DMA,
        pltpu.SemaphoreType.DMA,
        pltpu.SemaphoreType.DMA,
        pltpu.SemaphoreType.DMA,
        pltpu.SemaphoreType.DMA,
        pltpu.SemaphoreType.DMA,
    ],
)(_spmm3_body)


def kernel(user_emb, item_emb, edge_values, edge_index):
    all_emb = jnp.concatenate(
        [user_emb, item_emb, jnp.zeros((_NP - _N, 32), jnp.float32)], axis=0)
    x0 = jnp.stack([all_emb[:, :_H], all_emb[:, _H:]], axis=0)
    rows = edge_index[0].astype(jnp.int32)
    cols = edge_index[1].astype(jnp.int32)
    pad = _EPAD - _E
    cols_p = jnp.concatenate([cols, jnp.zeros((pad,), jnp.int32)]).reshape(-1, _SUB)
    rows_p = jnp.concatenate([rows, jnp.zeros((pad,), jnp.int32)]).reshape(-1, _SUB)
    vals_p = jnp.concatenate([edge_values, jnp.zeros((pad,), jnp.float32)]).reshape(-1, _SUB)
    vals_i = jax.lax.bitcast_convert_type(vals_p, jnp.int32)
    edata = jnp.stack([cols_p, rows_p, vals_i], axis=1)
    sum_o, _ = _spmm3(x0, edata)
    final = jnp.concatenate([sum_o[0, :_N], sum_o[1, :_N]], axis=1)
    return final[:_NUM_USERS], final[_NUM_USERS:]


# speedup vs baseline: 1.4430x; 1.0573x over previous
import functools

import jax
import jax.numpy as jnp
from jax import lax
from jax.experimental import pallas as pl
from jax.experimental.pallas import tpu as pltpu
from jax.experimental.pallas import tpu_sc as plsc

_NUM_USERS = 30000
_NUM_ITEMS = 70000
_N = _NUM_USERS + _NUM_ITEMS
_H = 16
_E = 1600000
_NS = 16
_SUB = 128
_CHUNK = 512
_K = _CHUNK // _SUB
_NMACRO = 66
_NCHUNK = 3 * _NMACRO
_EPT = _NCHUNK * _CHUNK
_EPAD = _EPT * _NS
_IROWS_PT = _EPT // _SUB
_NP = 100352
_RPT = _NP // _NS
_QROWS = 448
_NQ = _RPT // _QROWS


def _spmm3_body(x0, edata, sum_o, xbuf,
                g0, g1, g2, e0, e1, e2, acc,
                sg0, sg1, sg2, ss0, ss1, ss2):
    c = lax.axis_index("c")
    s = lax.axis_index("s")
    r0 = s * _RPT
    gs = (g0, g1, g2)
    es = (e0, e1, e2)
    sgs = (sg0, sg1, sg2)
    sss = (ss0, ss1, ss2)
    sa_v = g0.at[pl.ds(0, _QROWS)]
    sb_v = g1.at[pl.ds(0, _QROWS)]

    def load_e(i, p):
        pltpu.sync_copy(edata.at[pl.ds(s * _IROWS_PT + i * _K, _K)], es[p])

    def fire_gather(src, p):
        for j in range(_K):
            pltpu.async_copy(src.at[c].at[es[p].at[j, 0]],
                             gs[p].at[pl.ds(j * _SUB, _SUB)], sgs[p])

    def drain(src, sem):
        for j in range(_K):
            pltpu.make_async_copy(src.at[c].at[pl.ds(0, _SUB)],
                                  gs[0].at[pl.ds(0, _SUB)], sem).wait()

    def scale(p):
        @plsc.parallel_loop(0, _CHUNK // 16, unroll=2)
        def _(mg):
            m0 = mg * 16
            j = mg // (_SUB // 16)
            mm = (mg % (_SUB // 16)) * 16
            vv = plsc.bitcast(es[p][j, 2, pl.ds(mm, 16)], jnp.float32)
            for t in range(16):
                gs[p][m0 + t, :] = gs[p][m0 + t, :] * vv[t]

    def fire_scatter(p):
        for j in range(_K):
            pltpu.async_copy(gs[p].at[pl.ds(j * _SUB, _SUB)],
                             acc.at[es[p].at[j, 1]], sss[p], add=True)

    for l in range(3):
        def zrow(i, _):
            sa_v[i, :] = jnp.zeros((_H,), jnp.float32)
            return 0
        lax.fori_loop(0, _QROWS, zrow, 0)

        def zq(q, _):
            pltpu.sync_copy(sa_v, acc.at[pl.ds(r0 + q * _QROWS, _QROWS)])
            return 0
        lax.fori_loop(0, _NQ, zq, 0)
        plsc.subcore_barrier()

        src = x0 if l == 0 else xbuf

        load_e(0, 0)
        fire_gather(src, 0)

        def macro(t, _):
            for q in range(3):
                i = 3 * t + q
                pn = (q + 1) % 3
                @pl.when(i >= 2)
                def _():
                    drain(src, sss[pn])
                @pl.when(i + 1 < _NCHUNK)
                def _():
                    load_e(i + 1, pn)
                    fire_gather(src, pn)
                drain(src, sgs[q])
                scale(q)
                fire_scatter(q)
            return 0
        lax.fori_loop(0, _NMACRO, macro, 0)
        drain(src, sss[(_NCHUNK - 2) % 3])
        drain(src, sss[(_NCHUNK - 1) % 3])
        plsc.subcore_barrier()

        def cq(q, _):
            off = r0 + q * _QROWS
            pltpu.sync_copy(acc.at[pl.ds(off, _QROWS)], sa_v)
            if l < 2:
                pltpu.sync_copy(sa_v, xbuf.at[c].at[pl.ds(off, _QROWS)])
            prev = x0 if l == 0 else sum_o
            pltpu.sync_copy(prev.at[c].at[pl.ds(off, _QROWS)], sb_v)

            def addr(r, _):
                if l == 2:
                    sb_v[r, :] = (sb_v[r, :] + sa_v[r, :]) * 0.25
                else:
                    sb_v[r, :] = sb_v[r, :] + sa_v[r, :]
                return 0
            lax.fori_loop(0, _QROWS, addr, 0)
            pltpu.sync_copy(sb_v, sum_o.at[c].at[pl.ds(off, _QROWS)])
            return 0
        lax.fori_loop(0, _NQ, cq, 0)
        plsc.subcore_barrier()


_spmm3 = functools.partial(
    pl.kernel,
    mesh=plsc.VectorSubcoreMesh(core_axis_name="c", subcore_axis_name="s"),
    compiler_params=pltpu.CompilerParams(use_tc_tiling_on_sc=False,
                                         needs_layout_passes=False),
    out_type=[
        jax.ShapeDtypeStruct((2, _NP, _H), jnp.float32),
        jax.ShapeDtypeStruct((2, _NP, _H), jnp.float32),
    ],
    scratch_types=[
        pltpu.VMEM((_CHUNK, _H), jnp.float32),
        pltpu.VMEM((_CHUNK, _H), jnp.float32),
        pltpu.VMEM((_CHUNK, _H), jnp.float32),
        pltpu.VMEM((_K, 3, _SUB), jnp.int32),
        pltpu.VMEM((_K, 3, _SUB), jnp.int32),
        pltpu.VMEM((_K, 3, _SUB), jnp.int32),
        pltpu.VMEM_SHARED((_NP, _H), jnp.float32),
        pltpu.SemaphoreType.DMA,
        pltpu.SemaphoreType.DMA,
        pltpu.SemaphoreType.DMA,
        pltpu.SemaphoreType.DMA,
        pltpu.SemaphoreType.DMA,
        pltpu.SemaphoreType.DMA,
    ],
)(_spmm3_body)


def kernel(user_emb, item_emb, edge_values, edge_index):
    all_emb = jnp.concatenate(
        [user_emb, item_emb, jnp.zeros((_NP - _N, 32), jnp.float32)], axis=0)
    x0 = jnp.stack([all_emb[:, :_H], all_emb[:, _H:]], axis=0)
    rows = edge_index[0].astype(jnp.int32)
    cols = edge_index[1].astype(jnp.int32)
    pad = _EPAD - _E
    cols_p = jnp.concatenate([cols, jnp.zeros((pad,), jnp.int32)]).reshape(-1, _SUB)
    rows_p = jnp.concatenate([rows, jnp.zeros((pad,), jnp.int32)]).reshape(-1, _SUB)
    vals_p = jnp.concatenate([edge_values, jnp.zeros((pad,), jnp.float32)]).reshape(-1, _SUB)
    vals_i = jax.lax.bitcast_convert_type(vals_p, jnp.int32)
    edata = jnp.stack([cols_p, rows_p, vals_i], axis=1)
    sum_o, _ = _spmm3(x0, edata)
    final = jnp.concatenate([sum_o[0, :_N], sum_o[1, :_N]], axis=1)
    return final[:_NUM_USERS], final[_NUM_USERS:]


# speedup vs baseline: 1.4570x; 1.0097x over previous
import functools

import jax
import jax.numpy as jnp
from jax import lax
from jax.experimental import pallas as pl
from jax.experimental.pallas import tpu as pltpu
from jax.experimental.pallas import tpu_sc as plsc

_NUM_USERS = 30000
_NUM_ITEMS = 70000
_N = _NUM_USERS + _NUM_ITEMS
_H = 16
_E = 1600000
_NS = 16
_SUB = 128
_CHUNK = 512
_K = _CHUNK // _SUB
_NMACRO = 66
_NCHUNK = 3 * _NMACRO
_EPT = _NCHUNK * _CHUNK
_EPAD = _EPT * _NS
_IROWS_PT = _EPT // _SUB
_NP = 100352
_RPT = _NP // _NS
_QROWS = 448
_NQ = _RPT // _QROWS


def _spmm3_body(x0, edata, sum_o, xbuf,
                g0, g1, g2, e0, e1, e2, acc,
                sg0, sg1, sg2, ss0, ss1, ss2):
    c = lax.axis_index("c")
    s = lax.axis_index("s")
    r0 = s * _RPT
    gs = (g0, g1, g2)
    es = (e0, e1, e2)
    sgs = (sg0, sg1, sg2)
    sss = (ss0, ss1, ss2)
    sa_v = g0.at[pl.ds(0, _QROWS)]
    sb_v = g1.at[pl.ds(0, _QROWS)]

    def load_e(i, p):
        pltpu.sync_copy(edata.at[pl.ds(s * _IROWS_PT + i * _K, _K)], es[p])

    def fire_gather(src, p):
        for j in range(_K):
            pltpu.async_copy(src.at[c].at[es[p].at[j, 0]],
                             gs[p].at[pl.ds(j * _SUB, _SUB)], sgs[p])

    def drain(src, sem):
        for j in range(_K):
            pltpu.make_async_copy(src.at[c].at[pl.ds(0, _SUB)],
                                  gs[0].at[pl.ds(0, _SUB)], sem).wait()

    def scale(p):
        @plsc.parallel_loop(0, _CHUNK // 16, unroll=2)
        def _(mg):
            m0 = mg * 16
            j = mg // (_SUB // 16)
            mm = (mg % (_SUB // 16)) * 16
            vv = plsc.bitcast(es[p][j, 2, pl.ds(mm, 16)], jnp.float32)
            for t in range(16):
                gs[p][m0 + t, :] = gs[p][m0 + t, :] * vv[t]

    def fire_scatter(p):
        for j in range(_K):
            pltpu.async_copy(gs[p].at[pl.ds(j * _SUB, _SUB)],
                             acc.at[es[p].at[j, 1]], sss[p], add=True)

    for l in range(3):
        def zrow(i, _):
            sa_v[i, :] = jnp.zeros((_H,), jnp.float32)
            return 0
        lax.fori_loop(0, _QROWS, zrow, 0)

        def zq(q, _):
            pltpu.async_copy(sa_v, acc.at[pl.ds(r0 + q * _QROWS, _QROWS)], sg0)
            return 0
        lax.fori_loop(0, _NQ, zq, 0)

        def zqd(q, _):
            pltpu.make_async_copy(
                sa_v, acc.at[pl.ds(r0 + q * _QROWS, _QROWS)], sg0).wait()
            return 0
        lax.fori_loop(0, _NQ, zqd, 0)
        plsc.subcore_barrier()

        src = x0 if l == 0 else xbuf

        load_e(0, 0)
        fire_gather(src, 0)

        def macro(t, _):
            for q in range(3):
                i = 3 * t + q
                pn = (q + 1) % 3
                @pl.when(i >= 2)
                def _():
                    drain(src, sss[pn])
                @pl.when(i + 1 < _NCHUNK)
                def _():
                    load_e(i + 1, pn)
                    fire_gather(src, pn)
                drain(src, sgs[q])
                scale(q)
                fire_scatter(q)
            return 0
        lax.fori_loop(0, _NMACRO, macro, 0)
        drain(src, sss[(_NCHUNK - 2) % 3])
        drain(src, sss[(_NCHUNK - 1) % 3])
        plsc.subcore_barrier()

        def cq(q, _):
            off = r0 + q * _QROWS
            prev = x0 if l == 0 else sum_o
            pltpu.async_copy(acc.at[pl.ds(off, _QROWS)], sa_v, sg0)
            pltpu.async_copy(prev.at[c].at[pl.ds(off, _QROWS)], sb_v, sg1)
            pltpu.make_async_copy(acc.at[pl.ds(off, _QROWS)], sa_v, sg0).wait()
            pltpu.make_async_copy(prev.at[c].at[pl.ds(off, _QROWS)], sb_v,
                                  sg1).wait()
            if l < 2:
                pltpu.sync_copy(sa_v, xbuf.at[c].at[pl.ds(off, _QROWS)])

            def addr(r, _):
                if l == 2:
                    sb_v[r, :] = (sb_v[r, :] + sa_v[r, :]) * 0.25
                else:
                    sb_v[r, :] = sb_v[r, :] + sa_v[r, :]
                return 0
            lax.fori_loop(0, _QROWS, addr, 0)
            pltpu.sync_copy(sb_v, sum_o.at[c].at[pl.ds(off, _QROWS)])
            return 0
        lax.fori_loop(0, _NQ, cq, 0)
        plsc.subcore_barrier()


_spmm3 = functools.partial(
    pl.kernel,
    mesh=plsc.VectorSubcoreMesh(core_axis_name="c", subcore_axis_name="s"),
    compiler_params=pltpu.CompilerParams(use_tc_tiling_on_sc=False,
                                         needs_layout_passes=False),
    out_type=[
        jax.ShapeDtypeStruct((2, _NP, _H), jnp.float32),
        jax.ShapeDtypeStruct((2, _NP, _H), jnp.float32),
    ],
    scratch_types=[
        pltpu.VMEM((_CHUNK, _H), jnp.float32),
        pltpu.VMEM((_CHUNK, _H), jnp.float32),
        pltpu.VMEM((_CHUNK, _H), jnp.float32),
        pltpu.VMEM((_K, 3, _SUB), jnp.int32),
        pltpu.VMEM((_K, 3, _SUB), jnp.int32),
        pltpu.VMEM((_K, 3, _SUB), jnp.int32),
        pltpu.VMEM_SHARED((_NP, _H), jnp.float32),
        pltpu.SemaphoreType.DMA,
        pltpu.SemaphoreType.DMA,
        pltpu.SemaphoreType.DMA,
        pltpu.SemaphoreType.DMA,
        pltpu.SemaphoreType.DMA,
        pltpu.SemaphoreType.DMA,
    ],
)(_spmm3_body)


def kernel(user_emb, item_emb, edge_values, edge_index):
    all_emb = jnp.concatenate(
        [user_emb, item_emb, jnp.zeros((_NP - _N, 32), jnp.float32)], axis=0)
    x0 = jnp.stack([all_emb[:, :_H], all_emb[:, _H:]], axis=0)
    rows = edge_index[0].astype(jnp.int32)
    cols = edge_index[1].astype(jnp.int32)
    pad = _EPAD - _E
    cols_p = jnp.concatenate([cols, jnp.zeros((pad,), jnp.int32)]).reshape(-1, _SUB)
    rows_p = jnp.concatenate([rows, jnp.zeros((pad,), jnp.int32)]).reshape(-1, _SUB)
    vals_p = jnp.concatenate([edge_values, jnp.zeros((pad,), jnp.float32)]).reshape(-1, _SUB)
    vals_i = jax.lax.bitcast_convert_type(vals_p, jnp.int32)
    edata = jnp.stack([cols_p, rows_p, vals_i], axis=1)
    sum_o, _ = _spmm3(x0, edata)
    final = jnp.concatenate([sum_o[0, :_N], sum_o[1, :_N]], axis=1)
    return final[:_NUM_USERS], final[_NUM_USERS:]
